# baseline (speedup 1.0000x reference)
import functools

import jax
import jax.numpy as jnp
from jax import lax
from jax.experimental import pallas as pl
from jax.experimental.pallas import tpu as pltpu
from jax.experimental.pallas import tpu_sc as plsc

_TIMESTEPS = 1000
_N = 16384
_NUM_CORES = 1
_NUM_SUBCORES = 16
_NW = _NUM_CORES * _NUM_SUBCORES
_CHUNK = _N // _NW
_VEC = 16
_STEPS = _CHUNK // _VEC
_TBL = 1001


_MAGIC = jnp.float32(2.0**23)
_HALF = _CHUNK // 2


def _sc_body(t_hbm, g_hbm, out_hbm, t_v, g_v, o_v, idx_v, sem_g, sem_t):
    wid = lax.axis_index("s") * _NUM_CORES + lax.axis_index("c")
    base = wid * _CHUNK
    cg = pltpu.async_copy(g_hbm, g_v, sem_g)
    ct = pltpu.async_copy(t_hbm.at[pl.ds(base, _CHUNK)], t_v, sem_t)
    ct.wait()

    def comp(j, carry):
        off = j * _VEC
        tv = t_v[pl.ds(off, _VEC)]
        y = tv * jnp.float32(_TIMESTEPS)
        r = (y + _MAGIC) - _MAGIC
        idx_v[pl.ds(off, _VEC)] = jnp.minimum(
            jnp.maximum(r.astype(jnp.int32), 0), _TIMESTEPS
        )
        return carry

    lax.fori_loop(0, _STEPS, comp, 0, unroll=4)
    cg.wait()

    def gat(j, carry):
        off = j * _VEC
        idx = idx_v[pl.ds(off, _VEC)]
        o_v[pl.ds(off, _VEC)] = plsc.load_gather(g_v, [idx])
        return carry

    lax.fori_loop(0, _STEPS, gat, 0, unroll=4)
    pltpu.sync_copy(o_v, out_hbm.at[pl.ds(base, _CHUNK)])


@functools.cache
def _build_lookup():
    return functools.partial(
        pl.kernel,
        mesh=plsc.VectorSubcoreMesh(
            core_axis_name="c", subcore_axis_name="s", num_cores=_NUM_CORES
        ),
        out_type=jax.ShapeDtypeStruct((_N,), jnp.float32),
        scratch_types=[
            pltpu.VMEM((_CHUNK,), jnp.float32),
            pltpu.VMEM((_TBL,), jnp.float32),
            pltpu.VMEM((_CHUNK,), jnp.float32),
            pltpu.VMEM((_CHUNK,), jnp.int32),
            pltpu.SemaphoreType.DMA,
            pltpu.SemaphoreType.DMA,
        ],
        compiler_params=pltpu.CompilerParams(needs_layout_passes=False),
    )(_sc_body)


def kernel(t, gamma):
    tf = t.reshape(_N)
    out = _build_lookup()(tf, gamma)
    return out.reshape(t.shape)


# speedup vs baseline: 1.0276x; 1.0276x over previous
import functools

import jax
import jax.numpy as jnp
from jax import lax
from jax.experimental import pallas as pl
from jax.experimental.pallas import tpu as pltpu
from jax.experimental.pallas import tpu_sc as plsc

_TIMESTEPS = 1000
_N = 16384
_NUM_CORES = 1
_NUM_SUBCORES = 16
_NW = _NUM_CORES * _NUM_SUBCORES
_CHUNK = _N // _NW
_VEC = 16
_STEPS = _CHUNK // _VEC
_TBL = 1001

_MAGIC = 2.0**23


def _sc_body(t_hbm, g_hbm, out_hbm, t_v, g_v, o_v, sem_g, sem_t):
    wid = lax.axis_index("s") * _NUM_CORES + lax.axis_index("c")
    base = wid * _CHUNK
    cg = pltpu.async_copy(g_hbm, g_v, sem_g)
    ct = pltpu.async_copy(t_hbm.at[pl.ds(base, _CHUNK)], t_v, sem_t)
    ct.wait()
    cg.wait()

    def step(j, carry):
        off = j * _VEC
        tv = t_v[pl.ds(off, _VEC)]
        y = tv * jnp.float32(_TIMESTEPS)
        r = (y + jnp.float32(_MAGIC)) - jnp.float32(_MAGIC)
        idx = jnp.minimum(jnp.maximum(r.astype(jnp.int32), 0), _TIMESTEPS)
        o_v[pl.ds(off, _VEC)] = plsc.load_gather(g_v, [idx])
        return carry

    lax.fori_loop(0, _STEPS, step, 0, unroll=4)
    pltpu.sync_copy(o_v, out_hbm.at[pl.ds(base, _CHUNK)])


@functools.cache
def _build_lookup():
    return functools.partial(
        pl.kernel,
        mesh=plsc.VectorSubcoreMesh(
            core_axis_name="c", subcore_axis_name="s", num_cores=_NUM_CORES
        ),
        out_type=jax.ShapeDtypeStruct((_N,), jnp.float32),
        scratch_types=[
            pltpu.VMEM((_CHUNK,), jnp.float32),
            pltpu.VMEM((_TBL,), jnp.float32),
            pltpu.VMEM((_CHUNK,), jnp.float32),
            pltpu.SemaphoreType.DMA,
            pltpu.SemaphoreType.DMA,
        ],
        compiler_params=pltpu.CompilerParams(needs_layout_passes=False),
    )(_sc_body)


def kernel(t, gamma):
    tf = t.reshape(_N)
    out = _build_lookup()(tf, gamma)
    return out.reshape(t.shape)


# speedup vs baseline: 1.0763x; 1.0474x over previous
import functools

import jax
import jax.numpy as jnp
from jax import lax
from jax.experimental import pallas as pl
from jax.experimental.pallas import tpu as pltpu
from jax.experimental.pallas import tpu_sc as plsc

_TIMESTEPS = 1000
_N = 16384
_NUM_CORES = 1
_NUM_SUBCORES = 16
_NW = _NUM_CORES * _NUM_SUBCORES
_CHUNK = _N // _NW
_VEC = 16
_STEPS = _CHUNK // _VEC
_TBL = 1001

_MAGIC = 2.0**23


def _sc_body(t_hbm, g_hbm, out_hbm, t_v, g_v, o_v, sem_g, sem_t):
    wid = lax.axis_index("s") * _NUM_CORES + lax.axis_index("c")
    base = wid * _CHUNK
    cg = pltpu.async_copy(g_hbm, g_v, sem_g)
    ct = pltpu.async_copy(t_hbm.at[pl.ds(base, _CHUNK)], t_v, sem_t)
    ct.wait()
    cg.wait()

    @plsc.parallel_loop(0, _CHUNK, step=_VEC, unroll=4)
    def _loop(off):
        tv = t_v[pl.ds(off, _VEC)]
        y = tv * jnp.float32(_TIMESTEPS)
        r = (y + jnp.float32(_MAGIC)) - jnp.float32(_MAGIC)
        idx = jnp.minimum(jnp.maximum(r.astype(jnp.int32), 0), _TIMESTEPS)
        o_v[pl.ds(off, _VEC)] = plsc.load_gather(g_v, [idx])
    pltpu.sync_copy(o_v, out_hbm.at[pl.ds(base, _CHUNK)])


@functools.cache
def _build_lookup():
    return functools.partial(
        pl.kernel,
        mesh=plsc.VectorSubcoreMesh(
            core_axis_name="c", subcore_axis_name="s", num_cores=_NUM_CORES
        ),
        out_type=jax.ShapeDtypeStruct((_N,), jnp.float32),
        scratch_types=[
            pltpu.VMEM((_CHUNK,), jnp.float32),
            pltpu.VMEM((_TBL,), jnp.float32),
            pltpu.VMEM((_CHUNK,), jnp.float32),
            pltpu.SemaphoreType.DMA,
            pltpu.SemaphoreType.DMA,
        ],
        compiler_params=pltpu.CompilerParams(needs_layout_passes=False),
    )(_sc_body)


def kernel(t, gamma):
    tf = t.reshape(_N)
    out = _build_lookup()(tf, gamma)
    return out.reshape(t.shape)


# speedup vs baseline: 1.0811x; 1.0044x over previous
import functools

import jax
import jax.numpy as jnp
from jax import lax
from jax.experimental import pallas as pl
from jax.experimental.pallas import tpu as pltpu
from jax.experimental.pallas import tpu_sc as plsc

_TIMESTEPS = 1000
_N = 16384
_NUM_CORES = 1
_NUM_SUBCORES = 16
_NW = _NUM_CORES * _NUM_SUBCORES
_CHUNK = _N // _NW
_VEC = 16
_STEPS = _CHUNK // _VEC
_TBL = 1001

_MAGIC = 2.0**23


def _sc_body(t_hbm, g_hbm, out_hbm, t_v, g_v, o_v, sem_g, sem_t):
    wid = lax.axis_index("s") * _NUM_CORES + lax.axis_index("c")
    base = wid * _CHUNK
    cg = pltpu.async_copy(g_hbm, g_v, sem_g)
    ct = pltpu.async_copy(t_hbm.at[pl.ds(base, _CHUNK)], t_v, sem_t)
    ct.wait()
    cg.wait()

    @plsc.parallel_loop(0, _CHUNK, step=_VEC, unroll=8)
    def _loop(off):
        tv = t_v[pl.ds(off, _VEC)]
        y = tv * jnp.float32(_TIMESTEPS)
        r = (y + jnp.float32(_MAGIC)) - jnp.float32(_MAGIC)
        idx = jnp.minimum(jnp.maximum(r.astype(jnp.int32), 0), _TIMESTEPS)
        o_v[pl.ds(off, _VEC)] = plsc.load_gather(g_v, [idx])
    pltpu.sync_copy(o_v, out_hbm.at[pl.ds(base, _CHUNK)])


@functools.cache
def _build_lookup():
    return functools.partial(
        pl.kernel,
        mesh=plsc.VectorSubcoreMesh(
            core_axis_name="c", subcore_axis_name="s", num_cores=_NUM_CORES
        ),
        out_type=jax.ShapeDtypeStruct((_N,), jnp.float32),
        scratch_types=[
            pltpu.VMEM((_CHUNK,), jnp.float32),
            pltpu.VMEM((_TBL,), jnp.float32),
            pltpu.VMEM((_CHUNK,), jnp.float32),
            pltpu.SemaphoreType.DMA,
            pltpu.SemaphoreType.DMA,
        ],
        compiler_params=pltpu.CompilerParams(needs_layout_passes=False),
    )(_sc_body)


def kernel(t, gamma):
    tf = t.reshape(_N)
    out = _build_lookup()(tf, gamma)
    return out.reshape(t.shape)
